# baseline (speedup 1.0000x reference)
import functools

import jax
import jax.numpy as jnp
from jax import lax
from jax.experimental import pallas as pl
from jax.experimental.pallas import tpu as pltpu
from jax.experimental.pallas import tpu_sc as plsc

N = 10000
E = 160000
F = 256
S = 16
HALF = 128
ROW = S * HALF
ACC_W = 128
DROWS = 80
ACC_ROWS = 10240
EP = 163840
EPT = EP // 16
B = 32
SUP = 4
NSUP = EPT // (SUP * B)


_XW_ROWS = 400


NPART = 8
QROW = ROW // NPART


def _xw_body(x_ref, w_ref, o_ref):
    o_ref[0, 0] = jnp.dot(x_ref[...], w_ref[0, 0],
                          preferred_element_type=jnp.float32)


def _xw_table(x, wstk):
    return pl.pallas_call(
        _xw_body,
        grid=(2, NPART, N // _XW_ROWS),
        in_specs=[
            pl.BlockSpec((_XW_ROWS, F), lambda h, p, i: (i, 0)),
            pl.BlockSpec((1, 1, F, QROW), lambda h, p, i: (h, p, 0, 0)),
        ],
        out_specs=pl.BlockSpec((1, 1, _XW_ROWS, QROW),
                               lambda h, p, i: (h, p, i, 0)),
        out_shape=jax.ShapeDtypeStruct((2, NPART, N, QROW), jnp.float32),
    )(x, wstk)


def _sc_body(table, srcr, dstr, ear, eye, out, dout,
             idxa, idxb, dst_v, drow_v, lid_v, src_s, dst_s, ea_s,
             rowsa, rowsb, y_v, ydeg_v, bst, acc, dacc, sema, semb, esem):
    c = lax.axis_index("c")
    t = lax.axis_index("s")
    ebase = t * EPT
    rbase = t * (ACC_ROWS // 16)

    def _zrow(r, carry):
        for v in range(ACC_W // 16):
            y_v[r, pl.ds(v * 16, 16)] = jnp.zeros((16,), jnp.float32)
        return carry
    lax.fori_loop(0, B, _zrow, None)

    def _zacc(k, carry):
        pltpu.sync_copy(y_v, acc.at[pl.ds(rbase + k * B, B)])
        return carry
    lax.fori_loop(0, (ACC_ROWS // 16) // B, _zacc, None)

    @pl.when(t < DROWS // 8)
    def _():
        pltpu.sync_copy(y_v.at[pl.ds(0, 8)], dacc.at[pl.ds(t * 8, 8)])

    plsc.subcore_barrier()

    base_off = NPART * c * N

    def _pass(j, p, rbuf):
        def _edge(e, icarry):
            if p == 0:
                uv = ea_s[j * B + e]
                u0 = uv[0]
                u1 = uv[1]
                u2 = uv[2]
                u3 = uv[3]
                lane = lax.iota(jnp.int32, 16)
                b = jnp.where((lane & 1) != 0, u0, 1.0 - u0)
                b = b * jnp.where((lane & 2) != 0, u1, 1.0 - u1)
                b = b * jnp.where((lane & 4) != 0, u2, 1.0 - u2)
                b = b * jnp.where((lane & 8) != 0, u3, 1.0 - u3)
                bst[e, pl.ds(0, 16)] = b
            else:
                b = bst[e, pl.ds(0, 16)]
            ys = None
            for s in range(S):
                bs = b[s]
                rv = rbuf[e, pl.ds(s * 16, 16)]
                ys = bs * rv if s == 0 else ys + bs * rv
            y_v[e, pl.ds(p * 16, 16)] = ys
            return icarry
        lax.fori_loop(0, B, _edge, None)

    def _vset(dst_ref, src_ref, j, off):
        for k in range(B // 16):
            dst_ref[pl.ds(k * 16, 16)] = (
                src_ref[pl.ds(j * B + k * 16, 16)] + off)

    def _super(sb, carry):
        base = ebase + sb * (SUP * B)
        pltpu.sync_copy(srcr.at[pl.ds(base, SUP * B)], src_s)
        pltpu.sync_copy(dstr.at[pl.ds(base, SUP * B)], dst_s)
        pltpu.sync_copy(ear.at[pl.ds(base, SUP * B)], ea_s)
        _vset(idxa, src_s, 0, base_off)
        ga = pltpu.async_copy(table.at[idxa], rowsa, sema)
        for j in range(SUP):
            for k in range(B // 16):
                sl = pl.ds(k * 16, 16)
                dv = dst_s[pl.ds(j * B + k * 16, 16)]
                dst_v[sl] = dv
                drow_v[sl] = lax.shift_right_logical(dv, 7)
                lid_v[sl] = dv & 127
            deg_gat = pltpu.async_copy(eye.at[lid_v], ydeg_v, esem)
            _vset(idxb, src_s, j, base_off + N)
            gb = pltpu.async_copy(table.at[idxb], rowsb, semb)
            for p in range(NPART):
                if p % 2 == 0:
                    ga.wait()
                else:
                    gb.wait()
                _pass(j, p, rowsa if p % 2 == 0 else rowsb)
                if p + 2 < NPART:
                    if p % 2 == 0:
                        _vset(idxa, src_s, j, base_off + (p + 2) * N)
                        ga = pltpu.async_copy(table.at[idxa], rowsa, sema)
                    else:
                        _vset(idxb, src_s, j, base_off + (p + 2) * N)
                        gb = pltpu.async_copy(table.at[idxb], rowsb, semb)
                elif p == NPART - 2 and j + 1 < SUP:
                    _vset(idxa, src_s, j + 1, base_off)
                    ga = pltpu.async_copy(table.at[idxa], rowsa, sema)
            deg_gat.wait()
            pltpu.sync_copy(y_v, acc.at[dst_v], add=True)
            pltpu.sync_copy(ydeg_v, dacc.at[drow_v], add=True)
        return carry
    lax.fori_loop(0, NSUP, _super, None)

    plsc.subcore_barrier()
    pltpu.sync_copy(acc.at[pl.ds(rbase, ACC_ROWS // 16)],
                    out.at[pl.ds(c * ACC_ROWS + rbase, ACC_ROWS // 16)])

    @pl.when(t < DROWS // 8)
    def _():
        pltpu.sync_copy(dacc.at[pl.ds(t * 8, 8)],
                        dout.at[pl.ds(c * DROWS + t * 8, 8)])


def _sc_scatter(table, src_p, dst_p, ea_p):
    mesh = plsc.VectorSubcoreMesh(core_axis_name="c", subcore_axis_name="s")
    fn = functools.partial(
        pl.kernel,
        out_type=[
            jax.ShapeDtypeStruct((2 * ACC_ROWS, ACC_W), jnp.float32),
            jax.ShapeDtypeStruct((2 * DROWS, ACC_W), jnp.float32),
        ],
        mesh=mesh,
        scratch_types=[
            pltpu.VMEM((B,), jnp.int32),
            pltpu.VMEM((B,), jnp.int32),
            pltpu.VMEM((B,), jnp.int32),
            pltpu.VMEM((B,), jnp.int32),
            pltpu.VMEM((B,), jnp.int32),
            pltpu.VMEM((SUP * B,), jnp.int32),
            pltpu.VMEM((SUP * B,), jnp.int32),
            pltpu.VMEM((SUP * B, 16), jnp.float32),
            pltpu.VMEM((B, QROW), jnp.float32),
            pltpu.VMEM((B, QROW), jnp.float32),
            pltpu.VMEM((B, ACC_W), jnp.float32),
            pltpu.VMEM((B, ACC_W), jnp.float32),
            pltpu.VMEM((B, 16), jnp.float32),
            pltpu.VMEM_SHARED((ACC_ROWS, ACC_W), jnp.float32),
            pltpu.VMEM_SHARED((DROWS, ACC_W), jnp.float32),
            pltpu.SemaphoreType.DMA,
            pltpu.SemaphoreType.DMA,
            pltpu.SemaphoreType.DMA,
        ],
    )(_sc_body)
    eye = jnp.eye(ACC_W, dtype=jnp.float32)
    return fn(table, src_p, dst_p, ea_p, eye)


_EP_ROWS = 400


def _stats_body(accl_ref, acch_ref, deg_ref, x_ref, xs_ref, wr_ref, wl_ref,
                conv_ref, skip_ref, sums_ref):
    i = pl.program_id(0)
    feats = jnp.concatenate([accl_ref[...], acch_ref[...]], axis=1)
    deg = jnp.maximum(deg_ref[...], 1.0)
    conv = feats / deg + jnp.dot(x_ref[...], wr_ref[...],
                                 preferred_element_type=jnp.float32)
    skip = jnp.dot(xs_ref[...], wl_ref[...],
                   preferred_element_type=jnp.float32)
    conv_ref[...] = conv
    skip_ref[...] = skip

    @pl.when(i == 0)
    def _():
        sums_ref[...] = jnp.zeros_like(sums_ref)

    sums_ref[0:1, :] += jnp.sum(conv, axis=0, keepdims=True)
    sums_ref[1:2, :] += jnp.sum(conv * conv, axis=0, keepdims=True)
    sums_ref[2:3, :] += jnp.sum(skip, axis=0, keepdims=True)
    sums_ref[3:4, :] += jnp.sum(skip * skip, axis=0, keepdims=True)


def _stats(acc_lo, acc_hi, deg, x, x_skip, w_root, w_lin):
    nblk = N // _EP_ROWS
    return pl.pallas_call(
        _stats_body,
        grid=(nblk,),
        in_specs=[
            pl.BlockSpec((_EP_ROWS, ACC_W), lambda i: (i, 0)),
            pl.BlockSpec((_EP_ROWS, ACC_W), lambda i: (i, 0)),
            pl.BlockSpec((_EP_ROWS, 1), lambda i: (i, 0)),
            pl.BlockSpec((_EP_ROWS, F), lambda i: (i, 0)),
            pl.BlockSpec((_EP_ROWS, F), lambda i: (i, 0)),
            pl.BlockSpec((F, F), lambda i: (0, 0)),
            pl.BlockSpec((F, F), lambda i: (0, 0)),
        ],
        out_specs=[
            pl.BlockSpec((_EP_ROWS, F), lambda i: (i, 0)),
            pl.BlockSpec((_EP_ROWS, F), lambda i: (i, 0)),
            pl.BlockSpec((8, F), lambda i: (0, 0)),
        ],
        out_shape=[
            jax.ShapeDtypeStruct((N, F), jnp.float32),
            jax.ShapeDtypeStruct((N, F), jnp.float32),
            jax.ShapeDtypeStruct((8, F), jnp.float32),
        ],
    )(acc_lo, acc_hi, deg, x, x_skip, w_root, w_lin)


def _norm_body(conv_ref, skip_ref, sums_ref, g1_ref, b1_ref, g2_ref, b2_ref,
               o_ref):
    inv_n = 1.0 / N
    m1 = sums_ref[0:1, :] * inv_n
    v1 = sums_ref[1:2, :] * inv_n - m1 * m1
    m2 = sums_ref[2:3, :] * inv_n
    v2 = sums_ref[3:4, :] * inv_n - m2 * m2
    h = g1_ref[...] * (conv_ref[...] - m1) * lax.rsqrt(v1 + 1e-5) + b1_ref[...]
    hs = g2_ref[...] * (skip_ref[...] - m2) * lax.rsqrt(v2 + 1e-5) + b2_ref[...]
    z = h + hs
    o_ref[...] = jnp.where(z > 0, z, jnp.exp(z) - 1.0)


def _normalize(conv, skip, sums, g1, b1, g2, b2):
    nblk = N // _EP_ROWS
    vec = pl.BlockSpec((1, F), lambda i: (0, 0))
    return pl.pallas_call(
        _norm_body,
        grid=(nblk,),
        in_specs=[
            pl.BlockSpec((_EP_ROWS, F), lambda i: (i, 0)),
            pl.BlockSpec((_EP_ROWS, F), lambda i: (i, 0)),
            pl.BlockSpec((8, F), lambda i: (0, 0)),
            vec, vec, vec, vec,
        ],
        out_specs=pl.BlockSpec((_EP_ROWS, F), lambda i: (i, 0)),
        out_shape=jax.ShapeDtypeStruct((N, F), jnp.float32),
    )(conv, skip, sums, g1, b1, g2, b2)


def kernel(x, edge_index, edge_attr, x_skip, W_spline, W_root, W_lin,
           gamma1, beta1, gamma2, beta2):
    src = edge_index[0].astype(jnp.int32)
    dst = edge_index[1].astype(jnp.int32)
    pad = EP - E
    src_p = jnp.concatenate([src, jnp.zeros((pad,), jnp.int32)])
    dst_p = jnp.concatenate([dst, jnp.full((pad,), ACC_ROWS - 1, jnp.int32)])
    ea_p = jnp.pad(edge_attr.astype(jnp.float32), ((0, pad), (0, 12)))

    wt = W_spline.transpose(1, 0, 2)
    wstk = (jnp.stack([wt[:, :, :HALF].reshape(F, ROW),
                       wt[:, :, HALF:].reshape(F, ROW)])
            .reshape(2, F, S, NPART, 16).transpose(0, 3, 1, 2, 4)
            .reshape(2, NPART, F, QROW))

    table = _xw_table(x, wstk).reshape(2 * NPART * N, QROW)
    accs, degs = _sc_scatter(table, src_p, dst_p, ea_p)
    acc_lo = accs[:N]
    acc_hi = accs[ACC_ROWS:ACC_ROWS + N]
    deg = degs[:DROWS].reshape(ACC_ROWS)[:N].reshape(N, 1)

    conv, skip, sums = _stats(acc_lo, acc_hi, deg, x, x_skip, W_root, W_lin)
    return _normalize(conv, skip, sums,
                      gamma1.reshape(1, F), beta1.reshape(1, F),
                      gamma2.reshape(1, F), beta2.reshape(1, F))


# speedup vs baseline: 1.3455x; 1.3455x over previous
import functools

import jax
import jax.numpy as jnp
from jax import lax
from jax.experimental import pallas as pl
from jax.experimental.pallas import tpu as pltpu
from jax.experimental.pallas import tpu_sc as plsc

N = 10000
E = 160000
F = 256
S = 16
HALF = 128
ROW = S * HALF
ACC_W = 128
DROWS = 80
ACC_ROWS = 10240
EP = 163840
EPT = EP // 16
B = 16
SUP = 8
NSUP = EPT // (SUP * B)


_XW_ROWS = 400


NPART = 4
QROW = ROW // NPART


def _xw_body(x_ref, w_ref, o_ref):
    o_ref[0, 0] = jnp.dot(x_ref[...], w_ref[0, 0],
                          preferred_element_type=jnp.float32)


def _xw_table(x, wstk):
    return pl.pallas_call(
        _xw_body,
        grid=(2, NPART, N // _XW_ROWS),
        in_specs=[
            pl.BlockSpec((_XW_ROWS, F), lambda h, p, i: (i, 0)),
            pl.BlockSpec((1, 1, F, QROW), lambda h, p, i: (h, p, 0, 0)),
        ],
        out_specs=pl.BlockSpec((1, 1, _XW_ROWS, QROW),
                               lambda h, p, i: (h, p, i, 0)),
        out_shape=jax.ShapeDtypeStruct((2, NPART, N, QROW), jnp.float32),
    )(x, wstk)


def _sc_body(table, srcr, dstr, ear, eye, out, dout,
             idxa, idxb, dst_v, drow_v, lid_v, src_s, dst_s, ea_s,
             rowsa, rowsb, y_v, ydeg_v, bst, acc, dacc, sema, semb, esem):
    c = lax.axis_index("c")
    t = lax.axis_index("s")
    ebase = t * EPT
    rbase = t * (ACC_ROWS // 16)

    def _zrow(r, carry):
        for v in range(ACC_W // 16):
            y_v[r, pl.ds(v * 16, 16)] = jnp.zeros((16,), jnp.float32)
        return carry
    lax.fori_loop(0, B, _zrow, None)

    def _zacc(k, carry):
        pltpu.sync_copy(y_v, acc.at[pl.ds(rbase + k * B, B)])
        return carry
    lax.fori_loop(0, (ACC_ROWS // 16) // B, _zacc, None)

    @pl.when(t < DROWS // 8)
    def _():
        pltpu.sync_copy(y_v.at[pl.ds(0, 8)], dacc.at[pl.ds(t * 8, 8)])

    plsc.subcore_barrier()

    base_off = NPART * c * N

    def _pass(j, p, rbuf):
        def _edge(e, icarry):
            if p == 0:
                uv = ea_s[j * B + e]
                u0 = uv[0]
                u1 = uv[1]
                u2 = uv[2]
                u3 = uv[3]
                lane = lax.iota(jnp.int32, 16)
                b = jnp.where((lane & 1) != 0, u0, 1.0 - u0)
                b = b * jnp.where((lane & 2) != 0, u1, 1.0 - u1)
                b = b * jnp.where((lane & 4) != 0, u2, 1.0 - u2)
                b = b * jnp.where((lane & 8) != 0, u3, 1.0 - u3)
                bst[e, pl.ds(0, 16)] = b
            else:
                b = bst[e, pl.ds(0, 16)]
            ys = [None, None]
            for s in range(S):
                bs = b[s]
                for v in range(2):
                    rv = rbuf[e, pl.ds(s * 32 + v * 16, 16)]
                    if s == 0:
                        ys[v] = bs * rv
                    else:
                        ys[v] = ys[v] + bs * rv
            for v in range(2):
                y_v[e, pl.ds(p * 32 + v * 16, 16)] = ys[v]
            return icarry
        lax.fori_loop(0, B, _edge, None)

    def _vset(dst_ref, src_ref, j, off):
        for k in range(B // 16):
            dst_ref[pl.ds(k * 16, 16)] = (
                src_ref[pl.ds(j * B + k * 16, 16)] + off)

    def _super(sb, carry):
        base = ebase + sb * (SUP * B)
        pltpu.sync_copy(srcr.at[pl.ds(base, SUP * B)], src_s)
        pltpu.sync_copy(dstr.at[pl.ds(base, SUP * B)], dst_s)
        pltpu.sync_copy(ear.at[pl.ds(base, SUP * B)], ea_s)
        _vset(idxa, src_s, 0, base_off)
        ga = pltpu.async_copy(table.at[idxa], rowsa, sema)
        for j in range(SUP):
            for k in range(B // 16):
                sl = pl.ds(k * 16, 16)
                dv = dst_s[pl.ds(j * B + k * 16, 16)]
                dst_v[sl] = dv
                drow_v[sl] = lax.shift_right_logical(dv, 7)
                lid_v[sl] = dv & 127
            deg_gat = pltpu.async_copy(eye.at[lid_v], ydeg_v, esem)
            _vset(idxb, src_s, j, base_off + N)
            gb = pltpu.async_copy(table.at[idxb], rowsb, semb)
            for p in range(NPART):
                if p % 2 == 0:
                    ga.wait()
                else:
                    gb.wait()
                _pass(j, p, rowsa if p % 2 == 0 else rowsb)
                if p + 2 < NPART:
                    if p % 2 == 0:
                        _vset(idxa, src_s, j, base_off + (p + 2) * N)
                        ga = pltpu.async_copy(table.at[idxa], rowsa, sema)
                    else:
                        _vset(idxb, src_s, j, base_off + (p + 2) * N)
                        gb = pltpu.async_copy(table.at[idxb], rowsb, semb)
                elif p == NPART - 2 and j + 1 < SUP:
                    _vset(idxa, src_s, j + 1, base_off)
                    ga = pltpu.async_copy(table.at[idxa], rowsa, sema)
            deg_gat.wait()
            pltpu.sync_copy(y_v, acc.at[dst_v], add=True)
            pltpu.sync_copy(ydeg_v, dacc.at[drow_v], add=True)
        return carry
    lax.fori_loop(0, NSUP, _super, None)

    plsc.subcore_barrier()
    pltpu.sync_copy(acc.at[pl.ds(rbase, ACC_ROWS // 16)],
                    out.at[pl.ds(c * ACC_ROWS + rbase, ACC_ROWS // 16)])

    @pl.when(t < DROWS // 8)
    def _():
        pltpu.sync_copy(dacc.at[pl.ds(t * 8, 8)],
                        dout.at[pl.ds(c * DROWS + t * 8, 8)])


def _sc_scatter(table, src_p, dst_p, ea_p):
    mesh = plsc.VectorSubcoreMesh(core_axis_name="c", subcore_axis_name="s")
    fn = functools.partial(
        pl.kernel,
        out_type=[
            jax.ShapeDtypeStruct((2 * ACC_ROWS, ACC_W), jnp.float32),
            jax.ShapeDtypeStruct((2 * DROWS, ACC_W), jnp.float32),
        ],
        mesh=mesh,
        scratch_types=[
            pltpu.VMEM((B,), jnp.int32),
            pltpu.VMEM((B,), jnp.int32),
            pltpu.VMEM((B,), jnp.int32),
            pltpu.VMEM((B,), jnp.int32),
            pltpu.VMEM((B,), jnp.int32),
            pltpu.VMEM((SUP * B,), jnp.int32),
            pltpu.VMEM((SUP * B,), jnp.int32),
            pltpu.VMEM((SUP * B, 16), jnp.float32),
            pltpu.VMEM((B, QROW), jnp.float32),
            pltpu.VMEM((B, QROW), jnp.float32),
            pltpu.VMEM((B, ACC_W), jnp.float32),
            pltpu.VMEM((B, ACC_W), jnp.float32),
            pltpu.VMEM((B, 16), jnp.float32),
            pltpu.VMEM_SHARED((ACC_ROWS, ACC_W), jnp.float32),
            pltpu.VMEM_SHARED((DROWS, ACC_W), jnp.float32),
            pltpu.SemaphoreType.DMA,
            pltpu.SemaphoreType.DMA,
            pltpu.SemaphoreType.DMA,
        ],
    )(_sc_body)
    eye = jnp.eye(ACC_W, dtype=jnp.float32)
    return fn(table, src_p, dst_p, ea_p, eye)


_EP_ROWS = 400


def _stats_body(accl_ref, acch_ref, deg_ref, x_ref, xs_ref, wr_ref, wl_ref,
                conv_ref, skip_ref, sums_ref):
    i = pl.program_id(0)
    feats = jnp.concatenate([accl_ref[...], acch_ref[...]], axis=1)
    deg = jnp.maximum(deg_ref[...], 1.0)
    conv = feats / deg + jnp.dot(x_ref[...], wr_ref[...],
                                 preferred_element_type=jnp.float32)
    skip = jnp.dot(xs_ref[...], wl_ref[...],
                   preferred_element_type=jnp.float32)
    conv_ref[...] = conv
    skip_ref[...] = skip

    @pl.when(i == 0)
    def _():
        sums_ref[...] = jnp.zeros_like(sums_ref)

    sums_ref[0:1, :] += jnp.sum(conv, axis=0, keepdims=True)
    sums_ref[1:2, :] += jnp.sum(conv * conv, axis=0, keepdims=True)
    sums_ref[2:3, :] += jnp.sum(skip, axis=0, keepdims=True)
    sums_ref[3:4, :] += jnp.sum(skip * skip, axis=0, keepdims=True)


def _stats(acc_lo, acc_hi, deg, x, x_skip, w_root, w_lin):
    nblk = N // _EP_ROWS
    return pl.pallas_call(
        _stats_body,
        grid=(nblk,),
        in_specs=[
            pl.BlockSpec((_EP_ROWS, ACC_W), lambda i: (i, 0)),
            pl.BlockSpec((_EP_ROWS, ACC_W), lambda i: (i, 0)),
            pl.BlockSpec((_EP_ROWS, 1), lambda i: (i, 0)),
            pl.BlockSpec((_EP_ROWS, F), lambda i: (i, 0)),
            pl.BlockSpec((_EP_ROWS, F), lambda i: (i, 0)),
            pl.BlockSpec((F, F), lambda i: (0, 0)),
            pl.BlockSpec((F, F), lambda i: (0, 0)),
        ],
        out_specs=[
            pl.BlockSpec((_EP_ROWS, F), lambda i: (i, 0)),
            pl.BlockSpec((_EP_ROWS, F), lambda i: (i, 0)),
            pl.BlockSpec((8, F), lambda i: (0, 0)),
        ],
        out_shape=[
            jax.ShapeDtypeStruct((N, F), jnp.float32),
            jax.ShapeDtypeStruct((N, F), jnp.float32),
            jax.ShapeDtypeStruct((8, F), jnp.float32),
        ],
    )(acc_lo, acc_hi, deg, x, x_skip, w_root, w_lin)


def _norm_body(conv_ref, skip_ref, sums_ref, g1_ref, b1_ref, g2_ref, b2_ref,
               o_ref):
    inv_n = 1.0 / N
    m1 = sums_ref[0:1, :] * inv_n
    v1 = sums_ref[1:2, :] * inv_n - m1 * m1
    m2 = sums_ref[2:3, :] * inv_n
    v2 = sums_ref[3:4, :] * inv_n - m2 * m2
    h = g1_ref[...] * (conv_ref[...] - m1) * lax.rsqrt(v1 + 1e-5) + b1_ref[...]
    hs = g2_ref[...] * (skip_ref[...] - m2) * lax.rsqrt(v2 + 1e-5) + b2_ref[...]
    z = h + hs
    o_ref[...] = jnp.where(z > 0, z, jnp.exp(z) - 1.0)


def _normalize(conv, skip, sums, g1, b1, g2, b2):
    nblk = N // _EP_ROWS
    vec = pl.BlockSpec((1, F), lambda i: (0, 0))
    return pl.pallas_call(
        _norm_body,
        grid=(nblk,),
        in_specs=[
            pl.BlockSpec((_EP_ROWS, F), lambda i: (i, 0)),
            pl.BlockSpec((_EP_ROWS, F), lambda i: (i, 0)),
            pl.BlockSpec((8, F), lambda i: (0, 0)),
            vec, vec, vec, vec,
        ],
        out_specs=pl.BlockSpec((_EP_ROWS, F), lambda i: (i, 0)),
        out_shape=jax.ShapeDtypeStruct((N, F), jnp.float32),
    )(conv, skip, sums, g1, b1, g2, b2)


def kernel(x, edge_index, edge_attr, x_skip, W_spline, W_root, W_lin,
           gamma1, beta1, gamma2, beta2):
    src = edge_index[0].astype(jnp.int32)
    dst = edge_index[1].astype(jnp.int32)
    pad = EP - E
    src_p = jnp.concatenate([src, jnp.zeros((pad,), jnp.int32)])
    dst_p = jnp.concatenate([dst, jnp.full((pad,), ACC_ROWS - 1, jnp.int32)])
    ea_p = jnp.pad(edge_attr.astype(jnp.float32), ((0, pad), (0, 12)))

    wt = W_spline.transpose(1, 0, 2)
    wstk = (jnp.stack([wt[:, :, :HALF].reshape(F, ROW),
                       wt[:, :, HALF:].reshape(F, ROW)])
            .reshape(2, F, S, NPART, 32).transpose(0, 3, 1, 2, 4)
            .reshape(2, NPART, F, QROW))

    table = _xw_table(x, wstk).reshape(2 * NPART * N, QROW)
    accs, degs = _sc_scatter(table, src_p, dst_p, ea_p)
    acc_lo = accs[:N]
    acc_hi = accs[ACC_ROWS:ACC_ROWS + N]
    deg = degs[:DROWS].reshape(ACC_ROWS)[:N].reshape(N, 1)

    conv, skip, sums = _stats(acc_lo, acc_hi, deg, x, x_skip, W_root, W_lin)
    return _normalize(conv, skip, sums,
                      gamma1.reshape(1, F), beta1.reshape(1, F),
                      gamma2.reshape(1, F), beta2.reshape(1, F))
